# Initial kernel scaffold; baseline (speedup 1.0000x reference)
import functools

import jax
import jax.numpy as jnp
from jax import lax
from jax.experimental import pallas as pl
from jax.experimental.pallas import tpu as pltpu
from jax.experimental.pallas import tpu_sc as plsc

N = 320000
D = 128
S = 10000
NC = 2
NS = 16
NW = NC * NS
RT = N // NW
C = 128
NCHUNK = RT // C
TAIL = RT - NCHUNK * C
SEG_T = S // NS
CW = 16
ZR = 125

_LANES = 16


def _fill(ref, rows, cols, value):
    vec = jnp.full((_LANES,), value, jnp.float32)

    @pl.loop(0, rows)
    def _(r):
        for l in range(cols // _LANES):
            ref[r, pl.ds(l * _LANES, _LANES)] = vec


_sc_mesh = plsc.VectorSubcoreMesh(
    core_axis_name="c", subcore_axis_name="s", num_cores=NC, num_subcores=NS
)


@functools.partial(
    pl.kernel,
    out_type=(
        jax.ShapeDtypeStruct((NC, S, D), jnp.float32),
        jax.ShapeDtypeStruct((NC, S, CW), jnp.float32),
    ),
    mesh=_sc_mesh,
    scratch_types=[
        pltpu.VMEM_SHARED((S, D), jnp.float32),
        pltpu.VMEM_SHARED((S, CW), jnp.float32),
        pltpu.VMEM((C, D), jnp.float32),
        pltpu.VMEM((C, D), jnp.float32),
        pltpu.VMEM((C,), jnp.int32),
        pltpu.VMEM((C,), jnp.int32),
        pltpu.VMEM((C, CW), jnp.float32),
        pltpu.VMEM((C, D), jnp.float32),
        pltpu.VMEM((ZR, CW), jnp.float32),
        pltpu.VMEM((TAIL, D), jnp.float32),
        pltpu.VMEM((TAIL,), jnp.int32),
        pltpu.SemaphoreType.DMA,
        pltpu.SemaphoreType.DMA,
        pltpu.SemaphoreType.DMA,
        pltpu.SemaphoreType.DMA,
        pltpu.SemaphoreType.DMA,
        pltpu.SemaphoreType.DMA,
        pltpu.SemaphoreType.DMA,
        pltpu.SemaphoreType.DMA,
        pltpu.SemaphoreType.DMA,
    ],
)
def _sc_aggregate(x_hbm, ids_hbm, psum_hbm, pcnt_hbm,
                  acc_sum, acc_cnt, xa, xb, ia, ib, ones, zs, zc, xt, it,
                  gxa, gia, gxb, gib, sxa, sca, sxb, scb, wsem):
    cid = lax.axis_index("c")
    sid = lax.axis_index("s")
    row0 = (cid * NS + sid) * RT
    seg0 = sid * SEG_T

    def fire_gather(chunk, xbuf, ibuf, xsem, isem):
        r = row0 + chunk * C
        pltpu.async_copy(x_hbm.at[pl.ds(r, C)], xbuf, xsem)
        pltpu.async_copy(ids_hbm.at[pl.ds(r, C)], ibuf, isem)

    def wait_gather(chunk, xbuf, ibuf, xsem, isem):
        r = row0 + chunk * C
        pltpu.make_async_copy(x_hbm.at[pl.ds(r, C)], xbuf, xsem).wait()
        pltpu.make_async_copy(ids_hbm.at[pl.ds(r, C)], ibuf, isem).wait()

    fire_gather(0, xa, ia, gxa, gia)
    fire_gather(1, xb, ib, gxb, gib)

    _fill(zs, C, D, 0.0)
    _fill(zc, ZR, CW, 0.0)
    _fill(ones, C, CW, 1.0)

    zcopies = []
    for k in range(4):
        zcopies.append(pltpu.async_copy(
            zs, acc_sum.at[pl.ds(seg0 + k * C, C)], wsem))
    zcopies.append(pltpu.async_copy(
        zs.at[pl.ds(0, SEG_T - 4 * C)],
        acc_sum.at[pl.ds(seg0 + 4 * C, SEG_T - 4 * C)], wsem))
    for k in range(5):
        zcopies.append(pltpu.async_copy(
            zc, acc_cnt.at[pl.ds(seg0 + k * ZR, ZR)], wsem))
    for cp in zcopies:
        cp.wait()

    plsc.subcore_barrier()

    def scatter_chunk(xbuf, ibuf, xsem, csem):
        h1 = pltpu.async_copy(xbuf, acc_sum.at[ibuf], xsem, add=True)
        h2 = pltpu.async_copy(ones, acc_cnt.at[ibuf], csem, add=True)
        return h1, h2

    @pl.loop(0, NCHUNK // 2 - 1)
    def _(k):
        ca = 2 * k
        wait_gather(ca, xa, ia, gxa, gia)
        ha = scatter_chunk(xa, ia, sxa, sca)
        wait_gather(ca + 1, xb, ib, gxb, gib)
        hb = scatter_chunk(xb, ib, sxb, scb)
        ha[0].wait()
        ha[1].wait()
        fire_gather(ca + 2, xa, ia, gxa, gia)
        hb[0].wait()
        hb[1].wait()
        fire_gather(ca + 3, xb, ib, gxb, gib)

    wait_gather(NCHUNK - 2, xa, ia, gxa, gia)
    ha = scatter_chunk(xa, ia, sxa, sca)
    wait_gather(NCHUNK - 1, xb, ib, gxb, gib)
    hb = scatter_chunk(xb, ib, sxb, scb)
    ha[0].wait()
    ha[1].wait()
    hb[0].wait()
    hb[1].wait()

    rt = row0 + NCHUNK * C
    pltpu.sync_copy(x_hbm.at[pl.ds(rt, TAIL)], xt)
    pltpu.sync_copy(ids_hbm.at[pl.ds(rt, TAIL)], it)
    pltpu.sync_copy(xt, acc_sum.at[it], add=True)
    pltpu.sync_copy(ones.at[pl.ds(0, TAIL)], acc_cnt.at[it], add=True)

    plsc.subcore_barrier()

    w1 = pltpu.async_copy(
        acc_sum.at[pl.ds(seg0, SEG_T)], psum_hbm.at[cid, pl.ds(seg0, SEG_T)],
        wsem)
    w2 = pltpu.async_copy(
        acc_cnt.at[pl.ds(seg0, SEG_T)], pcnt_hbm.at[cid, pl.ds(seg0, SEG_T)],
        wsem)
    w1.wait()
    w2.wait()


_BLK = 1250


def _combine_body(ps_ref, pc_ref, o_ref):
    sums = ps_ref[0] + ps_ref[1]
    cnts = pc_ref[0] + pc_ref[1]
    cnt = jnp.maximum(cnts[:, 0:1], 1.0)
    o_ref[...] = sums / cnt


_combine = pl.pallas_call(
    _combine_body,
    grid=(S // _BLK,),
    in_specs=[
        pl.BlockSpec((NC, _BLK, D), lambda i: (0, i, 0)),
        pl.BlockSpec((NC, _BLK, CW), lambda i: (0, i, 0)),
    ],
    out_specs=pl.BlockSpec((_BLK, D), lambda i: (i, 0)),
    out_shape=jax.ShapeDtypeStruct((S, D), jnp.float32),
)


def kernel(x, segment_ids):
    psum, pcnt = _sc_aggregate(x, segment_ids)
    return _combine(psum, pcnt)


# speedup vs baseline: 8.6269x; 8.6269x over previous
import functools

import jax
import jax.numpy as jnp
from jax import lax
from jax.experimental import pallas as pl
from jax.experimental.pallas import tpu as pltpu
from jax.experimental.pallas import tpu_sc as plsc

N = 320000
D = 128
S = 10000
NC = 2
NS = 16
NW = NC * NS
RT = N // NW
C = 128
NCHUNK = RT // C
TAIL = RT - NCHUNK * C
WB = 624
WLAST = S - WB * (NS - 1)
CW = 16

_LANES = 16


def _fill(ref, rows, cols, value):
    vec = jnp.full((_LANES,), value, jnp.float32)

    @pl.loop(0, rows)
    def _(r):
        for l in range(cols // _LANES):
            ref[r, pl.ds(l * _LANES, _LANES)] = vec


_sc_mesh = plsc.VectorSubcoreMesh(
    core_axis_name="c", subcore_axis_name="s", num_cores=NC, num_subcores=NS
)


@functools.partial(
    pl.kernel,
    out_type=(
        jax.ShapeDtypeStruct((NC, S, D), jnp.float32),
        jax.ShapeDtypeStruct((NC, S, CW), jnp.float32),
    ),
    mesh=_sc_mesh,
    scratch_types=[
        pltpu.VMEM_SHARED((S, D), jnp.float32),
        pltpu.VMEM_SHARED((S, CW), jnp.float32),
        pltpu.VMEM((C, D), jnp.float32),
        pltpu.VMEM((C, D), jnp.float32),
        pltpu.VMEM((C,), jnp.int32),
        pltpu.VMEM((C,), jnp.int32),
        pltpu.VMEM((C, CW), jnp.float32),
        pltpu.VMEM((C, CW), jnp.float32),
        pltpu.VMEM((TAIL, D), jnp.float32),
        pltpu.VMEM((TAIL,), jnp.int32),
        pltpu.SemaphoreType.DMA,
        pltpu.SemaphoreType.DMA,
        pltpu.SemaphoreType.DMA,
        pltpu.SemaphoreType.DMA,
        pltpu.SemaphoreType.DMA,
        pltpu.SemaphoreType.DMA,
        pltpu.SemaphoreType.DMA,
        pltpu.SemaphoreType.DMA,
        pltpu.SemaphoreType.DMA,
    ],
    compiler_params=pltpu.CompilerParams(use_tc_tiling_on_sc=False),
)
def _sc_aggregate(x_hbm, ids_hbm, psum_hbm, pcnt_hbm,
                  acc_sum, acc_cnt, xa, xb, ia, ib, ones, zc, xt, it,
                  gxa, gia, gxb, gib, sxa, sca, sxb, scb, wsem):
    cid = lax.axis_index("c")
    sid = lax.axis_index("s")
    row0 = (cid * NS + sid) * RT
    base = sid * WB

    def fire_gather(chunk, xbuf, ibuf, xsem, isem):
        r = row0 + chunk * C
        pltpu.async_copy(x_hbm.at[pl.ds(r, C)], xbuf, xsem)
        pltpu.async_copy(ids_hbm.at[pl.ds(r, C)], ibuf, isem)

    def wait_gather(chunk, xbuf, ibuf, xsem, isem):
        r = row0 + chunk * C
        pltpu.make_async_copy(x_hbm.at[pl.ds(r, C)], xbuf, xsem).wait()
        pltpu.make_async_copy(ids_hbm.at[pl.ds(r, C)], ibuf, isem).wait()

    _fill(xa, C, D, 0.0)
    _fill(zc, C, CW, 0.0)
    _fill(ones, C, CW, 1.0)

    def zero_region(nfull, rem):
        cps = []
        for k in range(nfull):
            cps.append(pltpu.async_copy(
                xa, acc_sum.at[pl.ds(base + k * C, C)], wsem))
            cps.append(pltpu.async_copy(
                zc, acc_cnt.at[pl.ds(base + k * C, C)], wsem))
        if rem:
            cps.append(pltpu.async_copy(
                xa.at[pl.ds(0, rem)],
                acc_sum.at[pl.ds(base + nfull * C, rem)], wsem))
            cps.append(pltpu.async_copy(
                zc.at[pl.ds(0, rem)],
                acc_cnt.at[pl.ds(base + nfull * C, rem)], wsem))
        for cp in cps:
            cp.wait()

    @pl.when(sid < NS - 1)
    def _():
        zero_region(WB // C, WB % C)

    @pl.when(sid == NS - 1)
    def _():
        zero_region(WLAST // C, WLAST % C)

    fire_gather(0, xa, ia, gxa, gia)
    fire_gather(1, xb, ib, gxb, gib)

    plsc.subcore_barrier()

    def scatter_chunk(xbuf, ibuf, xsem, csem):
        h1 = pltpu.async_copy(xbuf, acc_sum.at[ibuf], xsem, add=True)
        h2 = pltpu.async_copy(ones, acc_cnt.at[ibuf], csem, add=True)
        return h1, h2

    @pl.loop(0, NCHUNK // 2 - 1)
    def _(k):
        ca = 2 * k
        wait_gather(ca, xa, ia, gxa, gia)
        ha = scatter_chunk(xa, ia, sxa, sca)
        wait_gather(ca + 1, xb, ib, gxb, gib)
        hb = scatter_chunk(xb, ib, sxb, scb)
        ha[0].wait()
        ha[1].wait()
        fire_gather(ca + 2, xa, ia, gxa, gia)
        hb[0].wait()
        hb[1].wait()
        fire_gather(ca + 3, xb, ib, gxb, gib)

    wait_gather(NCHUNK - 2, xa, ia, gxa, gia)
    ha = scatter_chunk(xa, ia, sxa, sca)
    wait_gather(NCHUNK - 1, xb, ib, gxb, gib)
    hb = scatter_chunk(xb, ib, sxb, scb)
    ha[0].wait()
    ha[1].wait()
    hb[0].wait()
    hb[1].wait()

    rt = row0 + NCHUNK * C
    pltpu.sync_copy(x_hbm.at[pl.ds(rt, TAIL)], xt)
    pltpu.sync_copy(ids_hbm.at[pl.ds(rt, TAIL)], it)
    pltpu.sync_copy(xt, acc_sum.at[it], add=True)
    pltpu.sync_copy(ones.at[pl.ds(0, TAIL)], acc_cnt.at[it], add=True)

    plsc.subcore_barrier()

    def write_out(nrows):
        w1 = pltpu.async_copy(
            acc_sum.at[pl.ds(base, nrows)],
            psum_hbm.at[cid, pl.ds(base, nrows)], wsem)
        w2 = pltpu.async_copy(
            acc_cnt.at[pl.ds(base, nrows)],
            pcnt_hbm.at[cid, pl.ds(base, nrows)], wsem)
        w1.wait()
        w2.wait()

    @pl.when(sid < NS - 1)
    def _():
        write_out(WB)

    @pl.when(sid == NS - 1)
    def _():
        write_out(WLAST)


_BLK = 1000


def _combine_body(ps_ref, pc_ref, o_ref):
    sums = ps_ref[0] + ps_ref[1]
    cnts = pc_ref[0] + pc_ref[1]
    cnt = jnp.maximum(cnts[:, 0:1], 1.0)
    o_ref[...] = sums / cnt


_combine = pl.pallas_call(
    _combine_body,
    grid=(S // _BLK,),
    in_specs=[
        pl.BlockSpec((NC, _BLK, D), lambda i: (0, i, 0)),
        pl.BlockSpec((NC, _BLK, CW), lambda i: (0, i, 0)),
    ],
    out_specs=pl.BlockSpec((_BLK, D), lambda i: (i, 0)),
    out_shape=jax.ShapeDtypeStruct((S, D), jnp.float32),
)


def kernel(x, segment_ids):
    psum, pcnt = _sc_aggregate(x, segment_ids)
    return _combine(psum, pcnt)
